# Initial kernel scaffold; baseline (speedup 1.0000x reference)
import jax
import jax.numpy as jnp
from jax.experimental import pallas as pl


def kernel(x, v, edge_index):
    raise NotImplementedError("write your pallas kernel here")


# speedup vs baseline: 51.7885x; 51.7885x over previous
import functools

import jax
import jax.numpy as jnp
from jax import lax
from jax.experimental import pallas as pl
from jax.experimental.pallas import tpu as pltpu
from jax.experimental.pallas import tpu_sc as plsc

N = 10000
D = 128
E = 320000
ONE_PLUS_EPS = 1.0 + 0.1

NC = 2
NS = 16
L = 16
NW = NC * NS
NP = 10240
SLICE = NP // NS
BATCH = 128
NBP = 79
EPW = NBP * BATCH
EP = NW * EPW

_MESH = plsc.VectorSubcoreMesh(core_axis_name="c", subcore_axis_name="s")


def _fill(ref, value, n):
    vec = jnp.full((L,), value, dtype=ref.dtype)
    for i in range(n // L):
        ref[pl.ds(i * L, L)] = vec


@functools.partial(
    pl.kernel,
    out_type=jax.ShapeDtypeStruct((NC, NP), jnp.float32),
    mesh=_MESH,
    compiler_params=pltpu.CompilerParams(needs_layout_passes=False),
    scratch_types=[
        pltpu.VMEM((NBP, BATCH), jnp.int32),
        pltpu.VMEM((BATCH,), jnp.float32),
        pltpu.VMEM((SLICE,), jnp.float32),
        pltpu.VMEM_SHARED((NP,), jnp.float32),
    ],
)
def _hist_kernel(srcp, out, src_v, ones_v, zbuf_v, acc_sp):
    c = lax.axis_index("c")
    s = lax.axis_index("s")
    wid = s * NC + c

    _fill(ones_v, 1.0, BATCH)
    _fill(zbuf_v, 0.0, SLICE)
    pltpu.sync_copy(zbuf_v, acc_sp.at[pl.ds(s * SLICE, SLICE)])
    pltpu.sync_copy(srcp.at[wid], src_v)
    plsc.subcore_barrier()

    def body(j, carry):
        pltpu.sync_copy(ones_v, acc_sp.at[src_v.at[j]], add=True)
        return carry

    lax.fori_loop(0, NBP, body, 0)
    plsc.subcore_barrier()
    pltpu.sync_copy(acc_sp.at[pl.ds(s * SLICE, SLICE)],
                    out.at[c, pl.ds(s * SLICE, SLICE)])


@functools.partial(
    pl.kernel,
    out_type=(jax.ShapeDtypeStruct((NP,), jnp.float32),
              jax.ShapeDtypeStruct((NC, NP), jnp.float32)),
    mesh=_MESH,
    compiler_params=pltpu.CompilerParams(needs_layout_passes=False),
    scratch_types=[
        pltpu.VMEM((NBP, BATCH), jnp.int32),
        pltpu.VMEM((NBP, BATCH), jnp.int32),
        pltpu.VMEM((NP,), jnp.float32),
        pltpu.VMEM((BATCH,), jnp.float32),
        pltpu.VMEM((SLICE,), jnp.float32),
        pltpu.VMEM((SLICE,), jnp.float32),
        pltpu.VMEM((SLICE,), jnp.float32),
        pltpu.VMEM((SLICE,), jnp.float32),
        pltpu.VMEM_SHARED((NP,), jnp.float32),
        pltpu.VMEM_SHARED((NP,), jnp.float32),
    ],
)
def _pass_kernel(wprev, accs, srcp, dstp, w_out, accs_out,
                 src_v, dst_v, w_v, vbuf_v, wp_v, a0_v, a1_v, comb_v,
                 w_sp, acc_sp):
    c = lax.axis_index("c")
    s = lax.axis_index("s")
    wid = s * NC + c
    sl = pl.ds(s * SLICE, SLICE)

    pltpu.sync_copy(wprev.at[sl], wp_v)
    pltpu.sync_copy(accs.at[0, sl], a0_v)
    pltpu.sync_copy(accs.at[1, sl], a1_v)
    for i in range(SLICE // L):
        ii = pl.ds(i * L, L)
        comb_v[ii] = ONE_PLUS_EPS * wp_v[ii] + a0_v[ii] + a1_v[ii]
    pltpu.sync_copy(comb_v, w_sp.at[sl])

    @pl.when(c == 0)
    def _():
        pltpu.sync_copy(comb_v, w_out.at[sl])

    _fill(wp_v, 0.0, SLICE)
    pltpu.sync_copy(wp_v, acc_sp.at[sl])

    pltpu.sync_copy(srcp.at[wid], src_v)
    pltpu.sync_copy(dstp.at[wid], dst_v)
    plsc.subcore_barrier()

    pltpu.sync_copy(w_sp, w_v)

    def body(j, carry):
        for t in range(BATCH // L):
            idx = dst_v[j, pl.ds(t * L, L)]
            vbuf_v[pl.ds(t * L, L)] = plsc.load_gather(w_v, [idx])
        pltpu.sync_copy(vbuf_v, acc_sp.at[src_v.at[j]], add=True)
        return carry

    lax.fori_loop(0, NBP, body, 0)
    plsc.subcore_barrier()
    pltpu.sync_copy(acc_sp.at[sl], accs_out.at[c, sl])


def _matvec_body(w2_ref, a0_ref, a1_ref, vp_ref, xp_ref, o_ref):
    u = ONE_PLUS_EPS * w2_ref[...] + a0_ref[...] + a1_ref[...]
    cw = u * vp_ref[...]
    o_ref[...] = jnp.dot(cw, xp_ref[...], preferred_element_type=jnp.float32)


_matvec = pl.pallas_call(
    _matvec_body,
    out_shape=jax.ShapeDtypeStruct((1, D), jnp.float32),
)


def kernel(x, v, edge_index):
    src = edge_index[0]
    dst = edge_index[1]
    pad = EP - E
    srcp = jnp.concatenate([src, jnp.full((pad,), N, jnp.int32)])
    dstp = jnp.concatenate([dst, jnp.zeros((pad,), jnp.int32)])
    srcp = srcp.reshape(NW, NBP, BATCH)
    dstp = dstp.reshape(NW, NBP, BATCH)

    ones = jnp.ones((NP,), jnp.float32)
    acc1 = _hist_kernel(srcp)
    w1, acc2 = _pass_kernel(ones, acc1, srcp, dstp)
    w2, acc3 = _pass_kernel(w1, acc2, srcp, dstp)

    vp = jnp.pad(v, (0, NP - N)).reshape(1, NP)
    xp = jnp.pad(x, ((0, NP - N), (0, 0)))
    q = _matvec(w2.reshape(1, NP), acc3[0].reshape(1, NP),
                acc3[1].reshape(1, NP), vp, xp)
    return q.reshape(D)


# speedup vs baseline: 64.5568x; 1.2465x over previous
import functools

import jax
import jax.numpy as jnp
from jax import lax
from jax.experimental import pallas as pl
from jax.experimental.pallas import tpu as pltpu
from jax.experimental.pallas import tpu_sc as plsc

N = 10000
D = 128
E = 320000
ONE_PLUS_EPS = 1.0 + 0.1

NC = 2
NS = 16
L = 16
NW = NC * NS
NP = 10240
SLICE = NP // NS
BATCH = 128
NBP = 79
EPW = NBP * BATCH
EP = NW * EPW

_MESH = plsc.VectorSubcoreMesh(core_axis_name="c", subcore_axis_name="s")


def _fill(ref, value, n):
    vec = jnp.full((L,), value, dtype=ref.dtype)
    for i in range(n // L):
        ref[pl.ds(i * L, L)] = vec


@functools.partial(
    pl.kernel,
    out_type=jax.ShapeDtypeStruct((NC, NP), jnp.float32),
    mesh=_MESH,
    compiler_params=pltpu.CompilerParams(needs_layout_passes=False),
    scratch_types=[
        pltpu.VMEM((NBP, BATCH), jnp.int32),
        pltpu.VMEM((BATCH,), jnp.float32),
        pltpu.VMEM((SLICE,), jnp.float32),
        pltpu.VMEM_SHARED((NP,), jnp.float32),
        pltpu.SemaphoreType.DMA,
    ],
)
def _hist_kernel(srcp, out, src_v, ones_v, zbuf_v, acc_sp, sem):
    c = lax.axis_index("c")
    s = lax.axis_index("s")
    wid = s * NC + c

    _fill(ones_v, 1.0, BATCH)
    _fill(zbuf_v, 0.0, SLICE)
    pltpu.sync_copy(zbuf_v, acc_sp.at[pl.ds(s * SLICE, SLICE)])
    pltpu.sync_copy(srcp.at[wid], src_v)
    plsc.subcore_barrier()

    def body(j, carry):
        pltpu.async_copy(ones_v, acc_sp.at[src_v.at[j]], sem, add=True)
        return carry

    lax.fori_loop(0, NBP, body, 0)

    def drain(j, carry):
        pltpu.make_async_copy(ones_v, acc_sp.at[src_v.at[j]], sem).wait()
        return carry

    lax.fori_loop(0, NBP, drain, 0)
    plsc.subcore_barrier()
    pltpu.sync_copy(acc_sp.at[pl.ds(s * SLICE, SLICE)],
                    out.at[c, pl.ds(s * SLICE, SLICE)])


@functools.partial(
    pl.kernel,
    out_type=(jax.ShapeDtypeStruct((NP,), jnp.float32),
              jax.ShapeDtypeStruct((NC, NP), jnp.float32)),
    mesh=_MESH,
    compiler_params=pltpu.CompilerParams(needs_layout_passes=False),
    scratch_types=[
        pltpu.VMEM((NBP, BATCH), jnp.int32),
        pltpu.VMEM((NBP, BATCH), jnp.int32),
        pltpu.VMEM((NP,), jnp.float32),
        pltpu.VMEM((NBP, BATCH), jnp.float32),
        pltpu.VMEM((SLICE,), jnp.float32),
        pltpu.VMEM((SLICE,), jnp.float32),
        pltpu.VMEM((SLICE,), jnp.float32),
        pltpu.VMEM((SLICE,), jnp.float32),
        pltpu.VMEM_SHARED((NP,), jnp.float32),
        pltpu.VMEM_SHARED((NP,), jnp.float32),
        pltpu.SemaphoreType.DMA,
    ],
)
def _pass_kernel(wprev, accs, srcp, dstp, w_out, accs_out,
                 src_v, dst_v, w_v, vals_v, wp_v, a0_v, a1_v, comb_v,
                 w_sp, acc_sp, sem):
    c = lax.axis_index("c")
    s = lax.axis_index("s")
    wid = s * NC + c
    sl = pl.ds(s * SLICE, SLICE)

    pltpu.sync_copy(wprev.at[sl], wp_v)
    pltpu.sync_copy(accs.at[0, sl], a0_v)
    pltpu.sync_copy(accs.at[1, sl], a1_v)
    for i in range(SLICE // L):
        ii = pl.ds(i * L, L)
        comb_v[ii] = ONE_PLUS_EPS * wp_v[ii] + a0_v[ii] + a1_v[ii]
    pltpu.sync_copy(comb_v, w_sp.at[sl])

    @pl.when(c == 0)
    def _():
        pltpu.sync_copy(comb_v, w_out.at[sl])

    _fill(wp_v, 0.0, SLICE)
    pltpu.sync_copy(wp_v, acc_sp.at[sl])

    pltpu.sync_copy(srcp.at[wid], src_v)
    pltpu.sync_copy(dstp.at[wid], dst_v)
    plsc.subcore_barrier()

    pltpu.sync_copy(w_sp, w_v)

    def body(j, carry):
        for t in range(BATCH // L):
            idx = dst_v[j, pl.ds(t * L, L)]
            vals_v[j, pl.ds(t * L, L)] = plsc.load_gather(w_v, [idx])
        pltpu.async_copy(vals_v.at[j], acc_sp.at[src_v.at[j]], sem, add=True)
        return carry

    lax.fori_loop(0, NBP, body, 0)

    def drain(j, carry):
        pltpu.make_async_copy(vals_v.at[j], acc_sp.at[src_v.at[j]], sem).wait()
        return carry

    lax.fori_loop(0, NBP, drain, 0)
    plsc.subcore_barrier()
    pltpu.sync_copy(acc_sp.at[sl], accs_out.at[c, sl])


def _matvec_body(w2_ref, a0_ref, a1_ref, vp_ref, x_ref, o_ref):
    u = ONE_PLUS_EPS * w2_ref[...] + a0_ref[...] + a1_ref[...]
    cw = (u * vp_ref[...])[:, :N]
    o_ref[...] = jnp.dot(cw, x_ref[...], preferred_element_type=jnp.float32)


_matvec = pl.pallas_call(
    _matvec_body,
    out_shape=jax.ShapeDtypeStruct((1, D), jnp.float32),
)


def kernel(x, v, edge_index):
    src = edge_index[0]
    dst = edge_index[1]
    pad = EP - E
    srcp = jnp.concatenate([src, jnp.full((pad,), N, jnp.int32)])
    dstp = jnp.concatenate([dst, jnp.zeros((pad,), jnp.int32)])
    srcp = srcp.reshape(NW, NBP, BATCH)
    dstp = dstp.reshape(NW, NBP, BATCH)

    ones = jnp.ones((NP,), jnp.float32)
    acc1 = _hist_kernel(srcp)
    w1, acc2 = _pass_kernel(ones, acc1, srcp, dstp)
    w2, acc3 = _pass_kernel(w1, acc2, srcp, dstp)

    vp = jnp.pad(v, (0, NP - N)).reshape(1, NP)
    q = _matvec(w2.reshape(1, NP), acc3[0].reshape(1, NP),
                acc3[1].reshape(1, NP), vp, x)
    return q.reshape(D)


# speedup vs baseline: 74.7077x; 1.1572x over previous
import functools

import jax
import jax.numpy as jnp
from jax import lax
from jax.experimental import pallas as pl
from jax.experimental.pallas import tpu as pltpu
from jax.experimental.pallas import tpu_sc as plsc

N = 10000
D = 128
E = 320000
ONE_PLUS_EPS = 1.0 + 0.1

NC = 2
NS = 16
L = 16
NW = NC * NS
NP = 10240
SLICE = NP // NS
BATCH = 128
NBP = 79
EP = NW * NBP * BATCH

_MESH = plsc.VectorSubcoreMesh(core_axis_name="c", subcore_axis_name="s")


def _fill(ref, value, n):
    vec = jnp.full((L,), value, dtype=ref.dtype)
    for i in range(n // L):
        ref[pl.ds(i * L, L)] = vec


@functools.partial(
    pl.kernel,
    out_type=(jax.ShapeDtypeStruct((NP,), jnp.float32),
              jax.ShapeDtypeStruct((3, NC, NP), jnp.float32)),
    mesh=_MESH,
    compiler_params=pltpu.CompilerParams(needs_layout_passes=False),
    scratch_types=[
        pltpu.VMEM((NBP, BATCH), jnp.int32),
        pltpu.VMEM((NBP, BATCH), jnp.int32),
        pltpu.VMEM((NP,), jnp.float32),
        pltpu.VMEM((NBP, BATCH), jnp.float32),
        pltpu.VMEM((SLICE,), jnp.float32),
        pltpu.VMEM((SLICE,), jnp.float32),
        pltpu.VMEM((SLICE,), jnp.float32),
        pltpu.VMEM((SLICE,), jnp.float32),
        pltpu.VMEM_SHARED((NP,), jnp.float32),
        pltpu.VMEM_SHARED((NP,), jnp.float32),
        pltpu.SemaphoreType.DMA,
        pltpu.SemaphoreType.REGULAR,
    ],
)
def _u_kernel(srcp, dstp, w_out, xchg,
              src_v, dst_v, w_v, vals_v, rem_v, loc_v, comb_v, zb_v,
              w_sp, acc_sp, dsem, xsem):
    c = lax.axis_index("c")
    s = lax.axis_index("s")
    wid = s * NC + c
    sl = pl.ds(s * SLICE, SLICE)

    def xbarrier():
        plsc.subcore_barrier()

        @pl.when(s == 0)
        def _():
            pltpu.semaphore_signal(xsem, 1, core_index=1 - c)
            pltpu.semaphore_wait(xsem, 1)

        plsc.subcore_barrier()

    def scatter_fire_drain():
        def body(j, carry):
            pltpu.async_copy(vals_v.at[j], acc_sp.at[src_v.at[j]], dsem,
                             add=True)
            return carry

        lax.fori_loop(0, NBP, body, 0)

        def drain(j, carry):
            pltpu.make_async_copy(vals_v.at[j], acc_sp.at[src_v.at[j]],
                                  dsem).wait()
            return carry

        lax.fori_loop(0, NBP, drain, 0)

    _fill(zb_v, 0.0, SLICE)
    pltpu.sync_copy(zb_v, acc_sp.at[sl])
    pltpu.sync_copy(srcp.at[wid], src_v)
    pltpu.sync_copy(dstp.at[wid], dst_v)
    _fill(vals_v.at[0], 1.0, BATCH)

    def ones_body(j, carry):
        pltpu.async_copy(vals_v.at[0], acc_sp.at[src_v.at[j]], dsem, add=True)
        return carry

    def ones_drain(j, carry):
        pltpu.make_async_copy(vals_v.at[0], acc_sp.at[src_v.at[j]],
                              dsem).wait()
        return carry

    plsc.subcore_barrier()
    lax.fori_loop(0, NBP, ones_body, 0)
    lax.fori_loop(0, NBP, ones_drain, 0)
    plsc.subcore_barrier()
    pltpu.sync_copy(acc_sp.at[sl], xchg.at[0, c, sl])
    xbarrier()

    for p in range(2):
        pltpu.sync_copy(xchg.at[p, 1 - c, sl], rem_v)
        pltpu.sync_copy(acc_sp.at[sl], loc_v)
        for i in range(SLICE // L):
            ii = pl.ds(i * L, L)
            if p == 0:
                wprev = ONE_PLUS_EPS
            else:
                wprev = ONE_PLUS_EPS * w_v[pl.ds(s * SLICE + i * L, L)]
            comb_v[ii] = wprev + loc_v[ii] + rem_v[ii]
        pltpu.sync_copy(comb_v, w_sp.at[sl])
        pltpu.sync_copy(zb_v, acc_sp.at[sl])
        if p == 1:
            @pl.when(c == 0)
            def _():
                pltpu.sync_copy(comb_v, w_out.at[sl])
        plsc.subcore_barrier()
        pltpu.sync_copy(w_sp, w_v)

        def gbody(j, carry):
            for t in range(BATCH // L):
                idx = dst_v[j, pl.ds(t * L, L)]
                vals_v[j, pl.ds(t * L, L)] = plsc.load_gather(w_v, [idx])
            pltpu.async_copy(vals_v.at[j], acc_sp.at[src_v.at[j]], dsem,
                             add=True)
            return carry

        lax.fori_loop(0, NBP, gbody, 0)

        def gdrain(j, carry):
            pltpu.make_async_copy(vals_v.at[j], acc_sp.at[src_v.at[j]],
                                  dsem).wait()
            return carry

        lax.fori_loop(0, NBP, gdrain, 0)
        plsc.subcore_barrier()
        pltpu.sync_copy(acc_sp.at[sl], xchg.at[p + 1, c, sl])
        if p == 0:
            xbarrier()


def _matvec_body(w2_ref, a0_ref, a1_ref, vp_ref, x_ref, o_ref):
    u = ONE_PLUS_EPS * w2_ref[...] + a0_ref[...] + a1_ref[...]
    cw = (u * vp_ref[...])[:, :N]
    o_ref[...] = jnp.dot(cw, x_ref[...], preferred_element_type=jnp.float32)


_matvec = pl.pallas_call(
    _matvec_body,
    out_shape=jax.ShapeDtypeStruct((1, D), jnp.float32),
)


def kernel(x, v, edge_index):
    src = edge_index[0]
    dst = edge_index[1]
    pad = EP - E
    srcp = jnp.concatenate([src, jnp.full((pad,), N, jnp.int32)])
    dstp = jnp.concatenate([dst, jnp.zeros((pad,), jnp.int32)])
    srcp = srcp.reshape(NW, NBP, BATCH)
    dstp = dstp.reshape(NW, NBP, BATCH)

    w2, xchg = _u_kernel(srcp, dstp)

    vp = jnp.pad(v, (0, NP - N)).reshape(1, NP)
    q = _matvec(w2.reshape(1, NP), xchg[2, 0].reshape(1, NP),
                xchg[2, 1].reshape(1, NP), vp, x)
    return q.reshape(D)
